# Initial kernel scaffold; baseline (speedup 1.0000x reference)
import jax
import jax.numpy as jnp
from jax.experimental import pallas as pl


def kernel(edge_index_pd, edge_index_dp, edge_index_pm, edge_index_mp, edge_label_index, params):
    raise NotImplementedError("write your pallas kernel here")


# speedup vs baseline: 1.7858x; 1.7858x over previous
import functools

import jax
import jax.numpy as jnp
from jax.experimental import pallas as pl


N_P, N_D, N_M = 50000, 10000, 20000


def _gat_fast(x_src, x_dst, ei, p, n_dst):
    src, dst = ei[0], ei[1]
    h_src = x_src @ p['W_src']
    a_src = h_src @ p['att_src']
    a_dst = x_dst @ (p['W_dst'] @ p['att_dst'])
    alpha = jax.nn.leaky_relu(a_src[src] + a_dst[dst], 0.2)
    ex = jnp.exp(alpha)
    den = jax.ops.segment_sum(ex, dst, num_segments=n_dst)
    num = jax.ops.segment_sum(h_src[src] * ex[:, None], dst, num_segments=n_dst)
    return num / (den[:, None] + 1e-16) + p['bias']


def _hetero(x, edges, pc):
    return {
        'patient': _gat_fast(x['disease'], x['patient'], edges['dp'], pc['dp'], N_P)
                   + _gat_fast(x['medicine'], x['patient'], edges['mp'], pc['mp'], N_P),
        'disease': _gat_fast(x['patient'], x['disease'], edges['pd'], pc['pd'], N_D),
        'medicine': _gat_fast(x['patient'], x['medicine'], edges['pm'], pc['pm'], N_M),
    }


def _score_body(sp_ref, sm_ref, b_ref, o_ref):
    o_ref[...] = sp_ref[...] + sm_ref[...] + b_ref[0]


def kernel(edge_index_pd, edge_index_dp, edge_index_pm, edge_index_mp, edge_label_index, params):
    edges = {'pd': edge_index_pd, 'dp': edge_index_dp, 'pm': edge_index_pm, 'mp': edge_index_mp}
    x = params['emb']
    h = _hetero(x, edges, params['conv1'])
    h = {k: jax.nn.relu(v) for k, v in h.items()}
    z = _hetero(h, edges, params['conv2'])
    row, col = edge_label_index[0], edge_label_index[1]
    s_p = z['patient'] @ params['lin_W'][:64, 0]
    s_m = z['medicine'] @ params['lin_W'][64:, 0]
    sp_g = s_p[row]
    sm_g = s_m[col]
    L = sp_g.shape[0]
    out = pl.pallas_call(
        _score_body,
        out_shape=jax.ShapeDtypeStruct((L,), jnp.float32),
    )(sp_g, sm_g, params['lin_b'])
    return out


# speedup vs baseline: 15.1547x; 8.4862x over previous
import functools

import jax
import jax.numpy as jnp
from jax import lax
from jax.experimental import pallas as pl
from jax.experimental.pallas import tpu as pltpu
from jax.experimental.pallas import tpu_sc as plsc

N_P, N_D, N_M = 50000, 10000, 20000
E = 800000
L_LBL = 100000
NC, NS = 2, 16
GB = 80
EPS = 1e-16
F32 = jnp.float32


def _cdiv(a, b):
    return (a + b - 1) // b


def _node_dense(x, src_params, dst_params):
    n = x.shape[0]
    B = 1000
    ns, nd = len(src_params), len(dst_params)

    def body(*refs):
        x_ref = refs[0]
        pr = refs[1:1 + 2 * (ns + nd)]
        out = refs[1 + 2 * (ns + nd):]
        xv = x_ref[...]
        for i in range(ns):
            w = pr[2 * i][...]
            att = pr[2 * i + 1][...]
            h = jnp.dot(xv, w, preferred_element_type=F32)
            out[3 * i][...] = h[:, :32]
            out[3 * i + 1][...] = h[:, 32:]
            out[3 * i + 2][...] = jnp.dot(h, att, preferred_element_type=F32)
        for j in range(nd):
            w = pr[2 * ns + 2 * j][...]
            att = pr[2 * ns + 2 * j + 1][...]
            h = jnp.dot(xv, w, preferred_element_type=F32)
            out[3 * ns + j][...] = jnp.dot(h, att, preferred_element_type=F32)

    row = lambda bs: pl.BlockSpec(bs, lambda i: (i, 0))
    full = lambda bs: pl.BlockSpec(bs, lambda i: (0, 0))
    in_specs = [row((B, 64))] + [full((64, 64)), full((64, 1))] * (ns + nd)
    out_specs = ([row((B, 32)), row((B, 32)), row((B, 1))] * ns
                 + [row((B, 1))] * nd)
    out_shape = ([jax.ShapeDtypeStruct((n, 32), F32),
                  jax.ShapeDtypeStruct((n, 32), F32),
                  jax.ShapeDtypeStruct((n, 1), F32)] * ns
                 + [jax.ShapeDtypeStruct((n, 1), F32)] * nd)
    args = [x]
    for p in src_params:
        args += [p['W_src'], p['att_src'].reshape(64, 1)]
    for p in dst_params:
        args += [p['W_dst'], p['att_dst'].reshape(64, 1)]
    return pl.pallas_call(
        body, grid=(n // B,), in_specs=in_specs, out_specs=out_specs,
        out_shape=out_shape)(*args)


def _combine(n, contribs, biases, relu=False, score_w=None, score_b=None):
    B = 1000
    k = len(contribs)

    def body(*refs):
        ins = refs[:-1]
        o_ref = refs[-1]
        o_lo = jnp.zeros((B, 32), F32)
        o_hi = jnp.zeros((B, 32), F32)
        for i in range(k):
            nlo, nhi, dlo, dhi = ins[4 * i:4 * i + 4]
            o_lo = o_lo + nlo[...] / (dlo[...] + EPS)
            o_hi = o_hi + nhi[...] / (dhi[...] + EPS)
        bias = ins[4 * k][...]
        for i in range(1, k):
            bias = bias + ins[4 * k + i][...]
        o_lo = o_lo + bias[:, :32]
        o_hi = o_hi + bias[:, 32:]
        if relu:
            o_ref[...] = jnp.concatenate(
                [jnp.maximum(o_lo, 0.0), jnp.maximum(o_hi, 0.0)], axis=1)
        else:
            w = ins[4 * k + k][...]
            b = ins[4 * k + k + 1][...]
            o_ref[...] = (jnp.dot(o_lo, w[:32], preferred_element_type=F32)
                          + jnp.dot(o_hi, w[32:], preferred_element_type=F32)
                          + b)

    row = lambda bs: pl.BlockSpec(bs, lambda i: (i, 0))
    full = lambda bs: pl.BlockSpec(bs, lambda i: (0, 0))
    in_specs = []
    args = []
    for (nlo, nhi, dlo, dhi) in contribs:
        in_specs += [row((B, 32)), row((B, 32)), row((B, 1)), row((B, 1))]
        args += [nlo, nhi, dlo.reshape(n, 1), dhi.reshape(n, 1)]
    for b in biases:
        in_specs.append(full((1, 64)))
        args.append(b.reshape(1, 64))
    if score_w is None:
        out_shape = jax.ShapeDtypeStruct((n, 64), F32)
        out_spec = row((B, 64))
    else:
        in_specs += [full((64, 1)), full((1, 1))]
        args += [score_w.reshape(64, 1), score_b.reshape(1, 1)]
        out_shape = jax.ShapeDtypeStruct((n, 1), F32)
        out_spec = row((B, 1))
    return pl.pallas_call(
        body, grid=(n // B,), in_specs=in_specs, out_specs=out_spec,
        out_shape=out_shape)(*args)


@functools.lru_cache(maxsize=None)
def _edge_pass(n_src, n_dst):
    ept = E // NS
    nblk = ept // GB
    nch = n_dst // GB
    mesh = plsc.VectorSubcoreMesh(core_axis_name="c", subcore_axis_name="s")

    @functools.partial(
        pl.kernel,
        out_type=[jax.ShapeDtypeStruct((n_dst, 32), F32),
                  jax.ShapeDtypeStruct((n_dst, 32), F32),
                  jax.ShapeDtypeStruct((n_dst,), F32),
                  jax.ShapeDtypeStruct((n_dst,), F32)],
        mesh=mesh,
        scratch_types=[
            pltpu.VMEM((GB,), jnp.int32),
            pltpu.VMEM((GB,), jnp.int32),
            pltpu.VMEM((GB, 32), F32),
            pltpu.VMEM((GB,), F32),
            pltpu.VMEM((GB,), F32),
            pltpu.VMEM((GB,), F32),
            pltpu.VMEM_SHARED((n_src,), F32),
            pltpu.VMEM_SHARED((n_dst,), F32),
            pltpu.VMEM_SHARED((n_dst, 32), F32),
            pltpu.VMEM_SHARED((n_dst,), F32),
            pltpu.SemaphoreType.DMA,
        ],
        compiler_params=pltpu.CompilerParams(needs_layout_passes=False,
                                             use_tc_tiling_on_sc=False),
    )
    def k(src_hbm, dst_hbm, asrc_hbm, adst_hbm, hlo_hbm, hhi_hbm,
          numlo_out, numhi_out, denlo_out, denhi_out,
          sidx, didx, hrows, exb, asb, adb, asrc_sh, adst_sh, num_sh, den_sh,
          sem):
        cid = lax.axis_index("c")
        sid = lax.axis_index("s")

        @pl.when(sid == 0)
        def _():
            pltpu.sync_copy(asrc_hbm, asrc_sh)
            pltpu.sync_copy(adst_hbm, adst_sh)

        def zrow(i, c):
            hrows[i, pl.ds(0, 16)] = jnp.zeros((16,), F32)
            hrows[i, pl.ds(16, 16)] = jnp.zeros((16,), F32)
            return c

        lax.fori_loop(0, GB, zrow, 0)
        for g in range(GB // 16):
            exb[pl.ds(g * 16, 16)] = jnp.zeros((16,), F32)

        def zchunk(j, c):
            ch = j * NS + sid

            @pl.when(ch < nch)
            def _():
                sl = pl.ds(ch * GB, GB)
                pltpu.sync_copy(hrows, num_sh.at[sl])
                pltpu.sync_copy(exb, den_sh.at[sl])
            return c

        lax.fori_loop(0, _cdiv(nch, NS), zchunk, 0)
        plsc.subcore_barrier()

        def eblk(b, c):
            off = sid * ept + b * GB
            pltpu.sync_copy(src_hbm.at[pl.ds(off, GB)], sidx)
            pltpu.sync_copy(dst_hbm.at[pl.ds(off, GB)], didx)

            @pl.when(cid == 0)
            def _():
                pltpu.async_copy(hlo_hbm.at[sidx], hrows, sem).wait()

            @pl.when(cid == 1)
            def _():
                pltpu.async_copy(hhi_hbm.at[sidx], hrows, sem).wait()

            pltpu.async_copy(asrc_sh.at[sidx], asb, sem).wait()
            pltpu.async_copy(adst_sh.at[didx], adb, sem).wait()

            for g in range(GB // 16):
                av = asb[pl.ds(g * 16, 16)] + adb[pl.ds(g * 16, 16)]
                av = jnp.where(av >= 0.0, av, av * 0.2)
                exb[pl.ds(g * 16, 16)] = jnp.exp(av)

            def srow(r, c2):
                w = plsc.load_gather(exb, [jnp.broadcast_to(r, (16,))])
                hrows[r, pl.ds(0, 16)] = hrows[r, pl.ds(0, 16)] * w
                hrows[r, pl.ds(16, 16)] = hrows[r, pl.ds(16, 16)] * w
                return c2

            lax.fori_loop(0, GB, srow, 0)
            pltpu.sync_copy(hrows, num_sh.at[didx], add=True)
            pltpu.sync_copy(exb, den_sh.at[didx], add=True)
            return c

        lax.fori_loop(0, nblk, eblk, 0)
        plsc.subcore_barrier()

        def rchunk(j, c):
            ch = j * NS + sid

            @pl.when(ch < nch)
            def _():
                sl = pl.ds(ch * GB, GB)

                @pl.when(cid == 0)
                def _():
                    pltpu.sync_copy(num_sh.at[sl], numlo_out.at[sl])
                    pltpu.sync_copy(den_sh.at[sl], denlo_out.at[sl])

                @pl.when(cid == 1)
                def _():
                    pltpu.sync_copy(num_sh.at[sl], numhi_out.at[sl])
                    pltpu.sync_copy(den_sh.at[sl], denhi_out.at[sl])
            return c

        lax.fori_loop(0, _cdiv(nch, NS), rchunk, 0)

    return k


@functools.lru_cache(maxsize=None)
def _score_kernel():
    nch = L_LBL // GB
    nw = NC * NS
    mesh = plsc.VectorSubcoreMesh(core_axis_name="c", subcore_axis_name="s")

    @functools.partial(
        pl.kernel,
        out_type=jax.ShapeDtypeStruct((L_LBL,), F32),
        mesh=mesh,
        scratch_types=[
            pltpu.VMEM((N_P,), F32),
            pltpu.VMEM((N_M,), F32),
            pltpu.VMEM((GB,), jnp.int32),
            pltpu.VMEM((GB,), jnp.int32),
            pltpu.VMEM((GB,), F32),
        ],
        compiler_params=pltpu.CompilerParams(needs_layout_passes=False),
    )
    def k(row_hbm, col_hbm, sp_hbm, sm_hbm, out_hbm, sp_t, sm_t, ridx, cidx, ob):
        wid = lax.axis_index("s") * NC + lax.axis_index("c")
        pltpu.sync_copy(sp_hbm, sp_t)
        pltpu.sync_copy(sm_hbm, sm_t)

        def chunk(j, c):
            ch = j * nw + wid

            @pl.when(ch < nch)
            def _():
                sl = pl.ds(ch * GB, GB)
                pltpu.sync_copy(row_hbm.at[sl], ridx)
                pltpu.sync_copy(col_hbm.at[sl], cidx)
                for g in range(GB // 16):
                    r16 = ridx[pl.ds(g * 16, 16)]
                    c16 = cidx[pl.ds(g * 16, 16)]
                    ob[pl.ds(g * 16, 16)] = (plsc.load_gather(sp_t, [r16])
                                             + plsc.load_gather(sm_t, [c16]))
                pltpu.sync_copy(ob, out_hbm.at[sl])
            return c

        lax.fori_loop(0, _cdiv(nch, nw), chunk, 0)

    return k


def _gat_layer(x, edges, pc, layer2):
    p_src_roles = [pc['pd'], pc['pm']] if not layer2 else [pc['pm']]
    p_dst_roles = [pc['dp'], pc['mp']]
    d_src_roles = [pc['dp']]
    d_dst_roles = [pc['pd']] if not layer2 else []
    m_src_roles = [pc['mp']]
    m_dst_roles = [pc['pm']]

    outp = _node_dense(x['patient'], p_src_roles, p_dst_roles)
    outd = _node_dense(x['disease'], d_src_roles, d_dst_roles)
    outm = _node_dense(x['medicine'], m_src_roles, m_dst_roles)

    if not layer2:
        h_pd = (outp[0], outp[1], outp[2])
        h_pm = (outp[3], outp[4], outp[5])
        a_dp_dst, a_mp_dst = outp[6], outp[7]
        h_dp = (outd[0], outd[1], outd[2])
        a_pd_dst = outd[3]
        h_mp = (outm[0], outm[1], outm[2])
        a_pm_dst = outm[3]
    else:
        h_pm = (outp[0], outp[1], outp[2])
        a_dp_dst, a_mp_dst = outp[3], outp[4]
        h_dp = (outd[0], outd[1], outd[2])
        h_mp = (outm[0], outm[1], outm[2])
        a_pm_dst = outm[3]

    def run(et, h, a_dst, n_src, n_dst):
        src = edges[et][0]
        dst = edges[et][1]
        return _edge_pass(n_src, n_dst)(
            src, dst, h[2].reshape(n_src), a_dst.reshape(n_dst), h[0], h[1])

    r_dp = run('dp', h_dp, a_dp_dst, N_D, N_P)
    r_mp = run('mp', h_mp, a_mp_dst, N_M, N_P)
    r_pm = run('pm', h_pm, a_pm_dst, N_P, N_M)
    r_pd = None if layer2 else run('pd', h_pd, a_pd_dst, N_P, N_D)
    return r_dp, r_mp, r_pm, r_pd


def kernel(edge_index_pd, edge_index_dp, edge_index_pm, edge_index_mp,
           edge_label_index, params):
    edges = {'pd': edge_index_pd, 'dp': edge_index_dp,
             'pm': edge_index_pm, 'mp': edge_index_mp}
    pc1, pc2 = params['conv1'], params['conv2']

    x0 = params['emb']
    r_dp, r_mp, r_pm, r_pd = _gat_layer(x0, edges, pc1, layer2=False)
    x1 = {
        'patient': _combine(N_P, [r_dp, r_mp],
                            [pc1['dp']['bias'], pc1['mp']['bias']], relu=True),
        'disease': _combine(N_D, [r_pd], [pc1['pd']['bias']], relu=True),
        'medicine': _combine(N_M, [r_pm], [pc1['pm']['bias']], relu=True),
    }

    r_dp2, r_mp2, r_pm2, _ = _gat_layer(x1, edges, pc2, layer2=True)
    w = params['lin_W']
    s_p = _combine(N_P, [r_dp2, r_mp2],
                   [pc2['dp']['bias'], pc2['mp']['bias']],
                   score_w=w[:64, 0], score_b=params['lin_b'])
    s_m = _combine(N_M, [r_pm2], [pc2['pm']['bias']],
                   score_w=w[64:, 0], score_b=jnp.zeros((1,), F32))

    return _score_kernel()(edge_label_index[0], edge_label_index[1],
                           s_p.reshape(N_P), s_m.reshape(N_M))


# speedup vs baseline: 30.9430x; 2.0418x over previous
import functools

import jax
import jax.numpy as jnp
from jax import lax
from jax.experimental import pallas as pl
from jax.experimental.pallas import tpu as pltpu
from jax.experimental.pallas import tpu_sc as plsc

N_P, N_D, N_M = 50000, 10000, 20000
E = 800000
L_LBL = 100000
NC, NS = 2, 16
GB = 80
EPS = 1e-16
F32 = jnp.float32


def _cdiv(a, b):
    return (a + b - 1) // b


def _node_dense(x, src_params, dst_params):
    n = x.shape[0]
    B = 1000
    ns, nd = len(src_params), len(dst_params)

    def body(*refs):
        x_ref = refs[0]
        pr = refs[1:1 + 2 * (ns + nd)]
        out = refs[1 + 2 * (ns + nd):]
        xv = x_ref[...]
        for i in range(ns):
            w = pr[2 * i][...]
            att = pr[2 * i + 1][...]
            h = jnp.dot(xv, w, preferred_element_type=F32)
            out[2 * i][0] = h[:, :32]
            out[2 * i][1] = h[:, 32:]
            out[2 * i + 1][...] = jnp.dot(h, att, preferred_element_type=F32)
        for j in range(nd):
            w = pr[2 * ns + 2 * j][...]
            att = pr[2 * ns + 2 * j + 1][...]
            h = jnp.dot(xv, w, preferred_element_type=F32)
            out[2 * ns + j][...] = jnp.dot(h, att, preferred_element_type=F32)

    row = lambda bs: pl.BlockSpec(bs, lambda i: (i, 0))
    row3 = pl.BlockSpec((2, B, 32), lambda i: (0, i, 0))
    full = lambda bs: pl.BlockSpec(bs, lambda i: (0, 0))
    in_specs = [row((B, 64))] + [full((64, 64)), full((64, 1))] * (ns + nd)
    out_specs = ([row3, row((B, 1))] * ns + [row((B, 1))] * nd)
    out_shape = ([jax.ShapeDtypeStruct((2, n, 32), F32),
                  jax.ShapeDtypeStruct((n, 1), F32)] * ns
                 + [jax.ShapeDtypeStruct((n, 1), F32)] * nd)
    args = [x]
    for p in src_params:
        args += [p['W_src'], p['att_src'].reshape(64, 1)]
    for p in dst_params:
        args += [p['W_dst'], p['att_dst'].reshape(64, 1)]
    return pl.pallas_call(
        body, grid=(n // B,), in_specs=in_specs, out_specs=out_specs,
        out_shape=out_shape)(*args)


def _combine(n, contribs, biases, relu=False, score_w=None, score_b=None):
    B = 1000
    k = len(contribs)

    def body(*refs):
        ins = refs[:-1]
        o_ref = refs[-1]
        o_lo = jnp.zeros((B, 32), F32)
        o_hi = jnp.zeros((B, 32), F32)
        for i in range(k):
            nlo, nhi, dlo, dhi = ins[4 * i:4 * i + 4]
            o_lo = o_lo + nlo[...] / (dlo[...] + EPS)
            o_hi = o_hi + nhi[...] / (dhi[...] + EPS)
        bias = ins[4 * k][...]
        for i in range(1, k):
            bias = bias + ins[4 * k + i][...]
        o_lo = o_lo + bias[:, :32]
        o_hi = o_hi + bias[:, 32:]
        if relu:
            o_ref[...] = jnp.concatenate(
                [jnp.maximum(o_lo, 0.0), jnp.maximum(o_hi, 0.0)], axis=1)
        else:
            w = ins[4 * k + k][...]
            b = ins[4 * k + k + 1][...]
            o_ref[...] = (jnp.dot(o_lo, w[:32], preferred_element_type=F32)
                          + jnp.dot(o_hi, w[32:], preferred_element_type=F32)
                          + b)

    row = lambda bs: pl.BlockSpec(bs, lambda i: (i, 0))
    full = lambda bs: pl.BlockSpec(bs, lambda i: (0, 0))
    in_specs = []
    args = []
    for (nlo, nhi, dlo, dhi) in contribs:
        in_specs += [row((B, 32)), row((B, 32)), row((B, 1)), row((B, 1))]
        args += [nlo, nhi, dlo.reshape(n, 1), dhi.reshape(n, 1)]
    for b in biases:
        in_specs.append(full((1, 64)))
        args.append(b.reshape(1, 64))
    if score_w is None:
        out_shape = jax.ShapeDtypeStruct((n, 64), F32)
        out_spec = row((B, 64))
    else:
        in_specs += [full((64, 1)), full((1, 1))]
        args += [score_w.reshape(64, 1), score_b.reshape(1, 1)]
        out_shape = jax.ShapeDtypeStruct((n, 1), F32)
        out_spec = row((B, 1))
    return pl.pallas_call(
        body, grid=(n // B,), in_specs=in_specs, out_specs=out_spec,
        out_shape=out_shape)(*args)


@functools.lru_cache(maxsize=None)
def _edge_pass(n_src, n_dst):
    ept = E // NS
    KR = 5
    BGB = KR * GB
    nblk = ept // BGB
    nch = n_dst // BGB
    mesh = plsc.VectorSubcoreMesh(core_axis_name="c", subcore_axis_name="s")

    @functools.partial(
        pl.kernel,
        out_type=[jax.ShapeDtypeStruct((n_dst, 32), F32),
                  jax.ShapeDtypeStruct((n_dst, 32), F32),
                  jax.ShapeDtypeStruct((n_dst,), F32),
                  jax.ShapeDtypeStruct((n_dst,), F32)],
        mesh=mesh,
        scratch_types=[
            pltpu.VMEM((BGB,), jnp.int32),
            pltpu.VMEM((BGB,), jnp.int32),
            pltpu.VMEM((BGB, 32), F32),
            pltpu.VMEM((BGB,), F32),
            pltpu.VMEM((BGB,), F32),
            pltpu.VMEM((BGB,), F32),
            pltpu.VMEM_SHARED((n_src,), F32),
            pltpu.VMEM_SHARED((n_dst,), F32),
            pltpu.VMEM_SHARED((n_dst, 32), F32),
            pltpu.VMEM_SHARED((n_dst,), F32),
            pltpu.SemaphoreType.DMA,
        ],
        compiler_params=pltpu.CompilerParams(needs_layout_passes=False,
                                             use_tc_tiling_on_sc=False),
    )
    def k(src2_hbm, dst2_hbm, asrc_hbm, adst_hbm, hlo_hbm, hhi_hbm,
          numlo_out, numhi_out, denlo_out, denhi_out,
          sidx, didx, hrows, exb, asb, adb,
          asrc_sh, adst_sh, num_sh, den_sh, sem):
        cid = lax.axis_index("c")
        sid = lax.axis_index("s")

        @pl.when(sid == 0)
        def _():
            pltpu.sync_copy(asrc_hbm, asrc_sh)
            pltpu.sync_copy(adst_hbm, adst_sh)

        def zrow(i, c):
            hrows[i, pl.ds(0, 16)] = jnp.zeros((16,), F32)
            hrows[i, pl.ds(16, 16)] = jnp.zeros((16,), F32)
            return c

        lax.fori_loop(0, BGB, zrow, 0)

        def zex(i, c):
            exb[pl.ds(i * 16, 16)] = jnp.zeros((16,), F32)
            return c

        lax.fori_loop(0, BGB // 16, zex, 0)

        def zchunk(j, c):
            ch = j * NS + sid

            @pl.when(ch < nch)
            def _():
                sl = pl.ds(ch * BGB, BGB)
                pltpu.sync_copy(hrows, num_sh.at[sl])
                pltpu.sync_copy(exb, den_sh.at[sl])
            return c

        lax.fori_loop(0, _cdiv(nch, NS), zchunk, 0)
        plsc.subcore_barrier()

        def eblk(b, c):
            off = sid * ept + b * BGB
            pltpu.sync_copy(src2_hbm.at[pl.ds(off, BGB)], sidx)
            pltpu.sync_copy(dst2_hbm.at[pl.ds(off, BGB)], didx)

            @pl.when(cid == 0)
            def _():
                pltpu.async_copy(hlo_hbm.at[sidx], hrows, sem).wait()

            @pl.when(cid == 1)
            def _():
                pltpu.async_copy(hhi_hbm.at[sidx], hrows, sem).wait()

            pltpu.async_copy(asrc_sh.at[sidx], asb, sem).wait()
            pltpu.async_copy(adst_sh.at[didx], adb, sem).wait()

            def cex(g, c2):
                av = asb[pl.ds(g * 16, 16)] + adb[pl.ds(g * 16, 16)]
                av = jnp.where(av >= 0.0, av, av * 0.2)
                exb[pl.ds(g * 16, 16)] = jnp.exp(av)
                return c2

            lax.fori_loop(0, BGB // 16, cex, 0)

            def srow(r, c2):
                w = plsc.load_gather(exb, [jnp.broadcast_to(r, (16,))])
                hrows[r, pl.ds(0, 16)] = hrows[r, pl.ds(0, 16)] * w
                hrows[r, pl.ds(16, 16)] = hrows[r, pl.ds(16, 16)] * w
                return c2

            lax.fori_loop(0, BGB, srow, 0)
            pltpu.sync_copy(hrows, num_sh.at[didx], add=True)
            pltpu.sync_copy(exb, den_sh.at[didx], add=True)
            return c

        lax.fori_loop(0, nblk, eblk, 0)
        plsc.subcore_barrier()

        def rchunk(j, c):
            ch = j * NS + sid

            @pl.when(ch < nch)
            def _():
                sl = pl.ds(ch * BGB, BGB)

                @pl.when(cid == 0)
                def _():
                    pltpu.sync_copy(num_sh.at[sl], numlo_out.at[sl])
                    pltpu.sync_copy(den_sh.at[sl], denlo_out.at[sl])

                @pl.when(cid == 1)
                def _():
                    pltpu.sync_copy(num_sh.at[sl], numhi_out.at[sl])
                    pltpu.sync_copy(den_sh.at[sl], denhi_out.at[sl])
            return c

        lax.fori_loop(0, _cdiv(nch, NS), rchunk, 0)

    return k


@functools.lru_cache(maxsize=None)
def _score_kernel():
    nch = L_LBL // GB
    nw = NC * NS
    mesh = plsc.VectorSubcoreMesh(core_axis_name="c", subcore_axis_name="s")

    @functools.partial(
        pl.kernel,
        out_type=jax.ShapeDtypeStruct((L_LBL,), F32),
        mesh=mesh,
        scratch_types=[
            pltpu.VMEM((N_P,), F32),
            pltpu.VMEM((N_M,), F32),
            pltpu.VMEM((GB,), jnp.int32),
            pltpu.VMEM((GB,), jnp.int32),
            pltpu.VMEM((GB,), F32),
        ],
        compiler_params=pltpu.CompilerParams(needs_layout_passes=False),
    )
    def k(row_hbm, col_hbm, sp_hbm, sm_hbm, out_hbm, sp_t, sm_t, ridx, cidx, ob):
        wid = lax.axis_index("s") * NC + lax.axis_index("c")
        pltpu.sync_copy(sp_hbm, sp_t)
        pltpu.sync_copy(sm_hbm, sm_t)

        def chunk(j, c):
            ch = j * nw + wid

            @pl.when(ch < nch)
            def _():
                sl = pl.ds(ch * GB, GB)
                pltpu.sync_copy(row_hbm.at[sl], ridx)
                pltpu.sync_copy(col_hbm.at[sl], cidx)
                for g in range(GB // 16):
                    r16 = ridx[pl.ds(g * 16, 16)]
                    c16 = cidx[pl.ds(g * 16, 16)]
                    ob[pl.ds(g * 16, 16)] = (plsc.load_gather(sp_t, [r16])
                                             + plsc.load_gather(sm_t, [c16]))
                pltpu.sync_copy(ob, out_hbm.at[sl])
            return c

        lax.fori_loop(0, _cdiv(nch, nw), chunk, 0)

    return k


def _gat_layer(x, edges, pc, layer2):
    p_src_roles = [pc['pd'], pc['pm']] if not layer2 else [pc['pm']]
    p_dst_roles = [pc['dp'], pc['mp']]
    d_src_roles = [pc['dp']]
    d_dst_roles = [pc['pd']] if not layer2 else []
    m_src_roles = [pc['mp']]
    m_dst_roles = [pc['pm']]

    outp = _node_dense(x['patient'], p_src_roles, p_dst_roles)
    outd = _node_dense(x['disease'], d_src_roles, d_dst_roles)
    outm = _node_dense(x['medicine'], m_src_roles, m_dst_roles)

    if not layer2:
        h_pd = (outp[0], outp[1])
        h_pm = (outp[2], outp[3])
        a_dp_dst, a_mp_dst = outp[4], outp[5]
        h_dp = (outd[0], outd[1])
        a_pd_dst = outd[2]
        h_mp = (outm[0], outm[1])
        a_pm_dst = outm[2]
    else:
        h_pm = (outp[0], outp[1])
        a_dp_dst, a_mp_dst = outp[2], outp[3]
        h_dp = (outd[0], outd[1])
        h_mp = (outm[0], outm[1])
        a_pm_dst = outm[2]

    def run(et, h, a_dst, n_src, n_dst):
        return _edge_pass(n_src, n_dst)(
            edges[et][0], edges[et][1], h[1].reshape(n_src),
            a_dst.reshape(n_dst), h[0][0], h[0][1])

    r_dp = run('dp', h_dp, a_dp_dst, N_D, N_P)
    r_mp = run('mp', h_mp, a_mp_dst, N_M, N_P)
    r_pm = run('pm', h_pm, a_pm_dst, N_P, N_M)
    r_pd = None if layer2 else run('pd', h_pd, a_pd_dst, N_P, N_D)
    return r_dp, r_mp, r_pm, r_pd


def kernel(edge_index_pd, edge_index_dp, edge_index_pm, edge_index_mp,
           edge_label_index, params):
    edges = {'pd': edge_index_pd, 'dp': edge_index_dp,
             'pm': edge_index_pm, 'mp': edge_index_mp}
    pc1, pc2 = params['conv1'], params['conv2']

    x0 = params['emb']
    r_dp, r_mp, r_pm, r_pd = _gat_layer(x0, edges, pc1, layer2=False)
    x1 = {
        'patient': _combine(N_P, [r_dp, r_mp],
                            [pc1['dp']['bias'], pc1['mp']['bias']], relu=True),
        'disease': _combine(N_D, [r_pd], [pc1['pd']['bias']], relu=True),
        'medicine': _combine(N_M, [r_pm], [pc1['pm']['bias']], relu=True),
    }

    r_dp2, r_mp2, r_pm2, _ = _gat_layer(x1, edges, pc2, layer2=True)
    w = params['lin_W']
    s_p = _combine(N_P, [r_dp2, r_mp2],
                   [pc2['dp']['bias'], pc2['mp']['bias']],
                   score_w=w[:64, 0], score_b=params['lin_b'])
    s_m = _combine(N_M, [r_pm2], [pc2['pm']['bias']],
                   score_w=w[64:, 0], score_b=jnp.zeros((1,), F32))

    return _score_kernel()(edge_label_index[0], edge_label_index[1],
                           s_p.reshape(N_P), s_m.reshape(N_M))


# speedup vs baseline: 36.7615x; 1.1880x over previous
import functools

import jax
import jax.numpy as jnp
from jax import lax
from jax.experimental import pallas as pl
from jax.experimental.pallas import tpu as pltpu
from jax.experimental.pallas import tpu_sc as plsc

N_P, N_D, N_M = 50000, 10000, 20000
E = 800000
L_LBL = 100000
NC, NS = 2, 16
GB = 80
EPS = 1e-16
F32 = jnp.float32


def _cdiv(a, b):
    return (a + b - 1) // b


def _node_dense(x, src_params, dst_params):
    n = x.shape[0]
    B = 1000
    ns, nd = len(src_params), len(dst_params)

    def body(*refs):
        x_ref = refs[0]
        pr = refs[1:1 + 2 * (ns + nd)]
        out = refs[1 + 2 * (ns + nd):]
        xv = x_ref[...]
        for i in range(ns):
            w = pr[2 * i][...]
            att = pr[2 * i + 1][...]
            h = jnp.dot(xv, w, preferred_element_type=F32)
            out[2 * i][0] = h[:, :32]
            out[2 * i][1] = h[:, 32:]
            out[2 * i + 1][...] = jnp.dot(h, att, preferred_element_type=F32)
        for j in range(nd):
            w = pr[2 * ns + 2 * j][...]
            att = pr[2 * ns + 2 * j + 1][...]
            h = jnp.dot(xv, w, preferred_element_type=F32)
            out[2 * ns + j][...] = jnp.dot(h, att, preferred_element_type=F32)

    row = lambda bs: pl.BlockSpec(bs, lambda i: (i, 0))
    row3 = pl.BlockSpec((2, B, 32), lambda i: (0, i, 0))
    full = lambda bs: pl.BlockSpec(bs, lambda i: (0, 0))
    in_specs = [row((B, 64))] + [full((64, 64)), full((64, 1))] * (ns + nd)
    out_specs = ([row3, row((B, 1))] * ns + [row((B, 1))] * nd)
    out_shape = ([jax.ShapeDtypeStruct((2, n, 32), F32),
                  jax.ShapeDtypeStruct((n, 1), F32)] * ns
                 + [jax.ShapeDtypeStruct((n, 1), F32)] * nd)
    args = [x]
    for p in src_params:
        args += [p['W_src'], p['att_src'].reshape(64, 1)]
    for p in dst_params:
        args += [p['W_dst'], p['att_dst'].reshape(64, 1)]
    return pl.pallas_call(
        body, grid=(n // B,), in_specs=in_specs, out_specs=out_specs,
        out_shape=out_shape)(*args)


def _combine(n, contribs, biases, relu=False, score_w=None, score_b=None):
    B = 1000
    k = len(contribs)

    def body(*refs):
        ins = refs[:-1]
        o_ref = refs[-1]
        o_lo = jnp.zeros((B, 32), F32)
        o_hi = jnp.zeros((B, 32), F32)
        for i in range(k):
            nlo, nhi, dlo, dhi = ins[4 * i:4 * i + 4]
            o_lo = o_lo + nlo[...] / (dlo[...] + EPS)
            o_hi = o_hi + nhi[...] / (dhi[...] + EPS)
        bias = ins[4 * k][...]
        for i in range(1, k):
            bias = bias + ins[4 * k + i][...]
        o_lo = o_lo + bias[:, :32]
        o_hi = o_hi + bias[:, 32:]
        if relu:
            o_ref[...] = jnp.concatenate(
                [jnp.maximum(o_lo, 0.0), jnp.maximum(o_hi, 0.0)], axis=1)
        else:
            w = ins[4 * k + k][...]
            b = ins[4 * k + k + 1][...]
            o_ref[...] = (jnp.dot(o_lo, w[:32], preferred_element_type=F32)
                          + jnp.dot(o_hi, w[32:], preferred_element_type=F32)
                          + b)

    row = lambda bs: pl.BlockSpec(bs, lambda i: (i, 0))
    full = lambda bs: pl.BlockSpec(bs, lambda i: (0, 0))
    in_specs = []
    args = []
    for (nlo, nhi, dlo, dhi) in contribs:
        in_specs += [row((B, 32)), row((B, 32)), row((B, 1)), row((B, 1))]
        args += [nlo, nhi, dlo.reshape(n, 1), dhi.reshape(n, 1)]
    for b in biases:
        in_specs.append(full((1, 64)))
        args.append(b.reshape(1, 64))
    if score_w is None:
        out_shape = jax.ShapeDtypeStruct((n, 64), F32)
        out_spec = row((B, 64))
    else:
        in_specs += [full((64, 1)), full((1, 1))]
        args += [score_w.reshape(64, 1), score_b.reshape(1, 1)]
        out_shape = jax.ShapeDtypeStruct((n, 1), F32)
        out_spec = row((B, 1))
    return pl.pallas_call(
        body, grid=(n // B,), in_specs=in_specs, out_specs=out_spec,
        out_shape=out_shape)(*args)


@functools.lru_cache(maxsize=None)
def _edge_pass(n_src, n_dst):
    ept = E // NS
    shared_words = 34 * n_dst + n_src
    BGB = 2000 if shared_words + 16 * 37 * 2000 < 2_000_000 else 400
    nblk = ept // BGB
    nch = n_dst // BGB
    mesh = plsc.VectorSubcoreMesh(core_axis_name="c", subcore_axis_name="s")

    @functools.partial(
        pl.kernel,
        out_type=[jax.ShapeDtypeStruct((n_dst, 32), F32),
                  jax.ShapeDtypeStruct((n_dst, 32), F32),
                  jax.ShapeDtypeStruct((n_dst,), F32),
                  jax.ShapeDtypeStruct((n_dst,), F32)],
        mesh=mesh,
        scratch_types=[
            pltpu.VMEM((BGB,), jnp.int32),
            pltpu.VMEM((BGB,), jnp.int32),
            pltpu.VMEM((BGB, 32), F32),
            pltpu.VMEM((BGB,), F32),
            pltpu.VMEM((BGB,), F32),
            pltpu.VMEM((BGB,), F32),
            pltpu.VMEM_SHARED((n_src,), F32),
            pltpu.VMEM_SHARED((n_dst,), F32),
            pltpu.VMEM_SHARED((n_dst, 32), F32),
            pltpu.VMEM_SHARED((n_dst,), F32),
            pltpu.SemaphoreType.DMA,
        ],
        compiler_params=pltpu.CompilerParams(needs_layout_passes=False,
                                             use_tc_tiling_on_sc=False),
    )
    def k(src2_hbm, dst2_hbm, asrc_hbm, adst_hbm, hlo_hbm, hhi_hbm,
          numlo_out, numhi_out, denlo_out, denhi_out,
          sidx, didx, hrows, exb, asb, adb,
          asrc_sh, adst_sh, num_sh, den_sh, sem):
        cid = lax.axis_index("c")
        sid = lax.axis_index("s")

        @pl.when(sid == 0)
        def _():
            pltpu.sync_copy(asrc_hbm, asrc_sh)
            pltpu.sync_copy(adst_hbm, adst_sh)

        def zrow(i, c):
            hrows[i, pl.ds(0, 16)] = jnp.zeros((16,), F32)
            hrows[i, pl.ds(16, 16)] = jnp.zeros((16,), F32)
            return c

        lax.fori_loop(0, BGB, zrow, 0)

        def zex(i, c):
            exb[pl.ds(i * 16, 16)] = jnp.zeros((16,), F32)
            return c

        lax.fori_loop(0, BGB // 16, zex, 0)

        def zchunk(j, c):
            ch = j * NS + sid

            @pl.when(ch < nch)
            def _():
                sl = pl.ds(ch * BGB, BGB)
                pltpu.sync_copy(hrows, num_sh.at[sl])
                pltpu.sync_copy(exb, den_sh.at[sl])
            return c

        lax.fori_loop(0, _cdiv(nch, NS), zchunk, 0)
        plsc.subcore_barrier()

        def eblk(b, c):
            off = sid * ept + b * BGB
            pltpu.sync_copy(src2_hbm.at[pl.ds(off, BGB)], sidx)
            pltpu.sync_copy(dst2_hbm.at[pl.ds(off, BGB)], didx)

            @pl.when(cid == 0)
            def _():
                pltpu.async_copy(hlo_hbm.at[sidx], hrows, sem).wait()

            @pl.when(cid == 1)
            def _():
                pltpu.async_copy(hhi_hbm.at[sidx], hrows, sem).wait()

            pltpu.async_copy(asrc_sh.at[sidx], asb, sem).wait()
            pltpu.async_copy(adst_sh.at[didx], adb, sem).wait()

            def cex(g, c2):
                av = asb[pl.ds(g * 16, 16)] + adb[pl.ds(g * 16, 16)]
                av = jnp.where(av >= 0.0, av, av * 0.2)
                exb[pl.ds(g * 16, 16)] = jnp.exp(av)
                return c2

            lax.fori_loop(0, BGB // 16, cex, 0)

            def srow(r0, c2):
                for u in range(4):
                    r = r0 * 4 + u
                    w = plsc.load_gather(exb, [jnp.broadcast_to(r, (16,))])
                    hrows[r, pl.ds(0, 16)] = hrows[r, pl.ds(0, 16)] * w
                    hrows[r, pl.ds(16, 16)] = hrows[r, pl.ds(16, 16)] * w
                return c2

            lax.fori_loop(0, BGB // 4, srow, 0)
            pltpu.sync_copy(hrows, num_sh.at[didx], add=True)
            pltpu.sync_copy(exb, den_sh.at[didx], add=True)
            return c

        lax.fori_loop(0, nblk, eblk, 0)
        plsc.subcore_barrier()

        def rchunk(j, c):
            ch = j * NS + sid

            @pl.when(ch < nch)
            def _():
                sl = pl.ds(ch * BGB, BGB)

                @pl.when(cid == 0)
                def _():
                    pltpu.sync_copy(num_sh.at[sl], numlo_out.at[sl])
                    pltpu.sync_copy(den_sh.at[sl], denlo_out.at[sl])

                @pl.when(cid == 1)
                def _():
                    pltpu.sync_copy(num_sh.at[sl], numhi_out.at[sl])
                    pltpu.sync_copy(den_sh.at[sl], denhi_out.at[sl])
            return c

        lax.fori_loop(0, _cdiv(nch, NS), rchunk, 0)

    return k


@functools.lru_cache(maxsize=None)
def _score_kernel():
    nch = L_LBL // GB
    nw = NC * NS
    mesh = plsc.VectorSubcoreMesh(core_axis_name="c", subcore_axis_name="s")

    @functools.partial(
        pl.kernel,
        out_type=jax.ShapeDtypeStruct((L_LBL,), F32),
        mesh=mesh,
        scratch_types=[
            pltpu.VMEM((N_P,), F32),
            pltpu.VMEM((N_M,), F32),
            pltpu.VMEM((GB,), jnp.int32),
            pltpu.VMEM((GB,), jnp.int32),
            pltpu.VMEM((GB,), F32),
        ],
        compiler_params=pltpu.CompilerParams(needs_layout_passes=False),
    )
    def k(row_hbm, col_hbm, sp_hbm, sm_hbm, out_hbm, sp_t, sm_t, ridx, cidx, ob):
        wid = lax.axis_index("s") * NC + lax.axis_index("c")
        pltpu.sync_copy(sp_hbm, sp_t)
        pltpu.sync_copy(sm_hbm, sm_t)

        def chunk(j, c):
            ch = j * nw + wid

            @pl.when(ch < nch)
            def _():
                sl = pl.ds(ch * GB, GB)
                pltpu.sync_copy(row_hbm.at[sl], ridx)
                pltpu.sync_copy(col_hbm.at[sl], cidx)
                for g in range(GB // 16):
                    r16 = ridx[pl.ds(g * 16, 16)]
                    c16 = cidx[pl.ds(g * 16, 16)]
                    ob[pl.ds(g * 16, 16)] = (plsc.load_gather(sp_t, [r16])
                                             + plsc.load_gather(sm_t, [c16]))
                pltpu.sync_copy(ob, out_hbm.at[sl])
            return c

        lax.fori_loop(0, _cdiv(nch, nw), chunk, 0)

    return k


def _gat_layer(x, edges, pc, layer2):
    p_src_roles = [pc['pd'], pc['pm']] if not layer2 else [pc['pm']]
    p_dst_roles = [pc['dp'], pc['mp']]
    d_src_roles = [pc['dp']]
    d_dst_roles = [pc['pd']] if not layer2 else []
    m_src_roles = [pc['mp']]
    m_dst_roles = [pc['pm']]

    outp = _node_dense(x['patient'], p_src_roles, p_dst_roles)
    outd = _node_dense(x['disease'], d_src_roles, d_dst_roles)
    outm = _node_dense(x['medicine'], m_src_roles, m_dst_roles)

    if not layer2:
        h_pd = (outp[0], outp[1])
        h_pm = (outp[2], outp[3])
        a_dp_dst, a_mp_dst = outp[4], outp[5]
        h_dp = (outd[0], outd[1])
        a_pd_dst = outd[2]
        h_mp = (outm[0], outm[1])
        a_pm_dst = outm[2]
    else:
        h_pm = (outp[0], outp[1])
        a_dp_dst, a_mp_dst = outp[2], outp[3]
        h_dp = (outd[0], outd[1])
        h_mp = (outm[0], outm[1])
        a_pm_dst = outm[2]

    def run(et, h, a_dst, n_src, n_dst):
        return _edge_pass(n_src, n_dst)(
            edges[et][0], edges[et][1], h[1].reshape(n_src),
            a_dst.reshape(n_dst), h[0][0], h[0][1])

    r_dp = run('dp', h_dp, a_dp_dst, N_D, N_P)
    r_mp = run('mp', h_mp, a_mp_dst, N_M, N_P)
    r_pm = run('pm', h_pm, a_pm_dst, N_P, N_M)
    r_pd = None if layer2 else run('pd', h_pd, a_pd_dst, N_P, N_D)
    return r_dp, r_mp, r_pm, r_pd


def kernel(edge_index_pd, edge_index_dp, edge_index_pm, edge_index_mp,
           edge_label_index, params):
    edges = {'pd': edge_index_pd, 'dp': edge_index_dp,
             'pm': edge_index_pm, 'mp': edge_index_mp}
    pc1, pc2 = params['conv1'], params['conv2']

    x0 = params['emb']
    r_dp, r_mp, r_pm, r_pd = _gat_layer(x0, edges, pc1, layer2=False)
    x1 = {
        'patient': _combine(N_P, [r_dp, r_mp],
                            [pc1['dp']['bias'], pc1['mp']['bias']], relu=True),
        'disease': _combine(N_D, [r_pd], [pc1['pd']['bias']], relu=True),
        'medicine': _combine(N_M, [r_pm], [pc1['pm']['bias']], relu=True),
    }

    r_dp2, r_mp2, r_pm2, _ = _gat_layer(x1, edges, pc2, layer2=True)
    w = params['lin_W']
    s_p = _combine(N_P, [r_dp2, r_mp2],
                   [pc2['dp']['bias'], pc2['mp']['bias']],
                   score_w=w[:64, 0], score_b=params['lin_b'])
    s_m = _combine(N_M, [r_pm2], [pc2['pm']['bias']],
                   score_w=w[64:, 0], score_b=jnp.zeros((1,), F32))

    return _score_kernel()(edge_label_index[0], edge_label_index[1],
                           s_p.reshape(N_P), s_m.reshape(N_M))


# speedup vs baseline: 41.3901x; 1.1259x over previous
import functools

import jax
import jax.numpy as jnp
from jax import lax
from jax.experimental import pallas as pl
from jax.experimental.pallas import tpu as pltpu
from jax.experimental.pallas import tpu_sc as plsc

N_P, N_D, N_M = 50000, 10000, 20000
E = 800000
L_LBL = 100000
NC, NS = 2, 16
GB = 80
EPS = 1e-16
F32 = jnp.float32


def _cdiv(a, b):
    return (a + b - 1) // b


def _node_dense(x, src_params, dst_params):
    n = x.shape[0]
    B = 1000
    ns, nd = len(src_params), len(dst_params)

    def body(*refs):
        x_ref = refs[0]
        pr = refs[1:1 + 2 * (ns + nd)]
        out = refs[1 + 2 * (ns + nd):]
        xv = x_ref[...]
        for i in range(ns):
            w = pr[2 * i][...]
            att = pr[2 * i + 1][...]
            h = jnp.dot(xv, w, preferred_element_type=F32)
            out[2 * i][0] = h[:, :32]
            out[2 * i][1] = h[:, 32:]
            out[2 * i + 1][...] = jnp.dot(h, att, preferred_element_type=F32)
        for j in range(nd):
            w = pr[2 * ns + 2 * j][...]
            att = pr[2 * ns + 2 * j + 1][...]
            h = jnp.dot(xv, w, preferred_element_type=F32)
            out[2 * ns + j][...] = jnp.dot(h, att, preferred_element_type=F32)

    row = lambda bs: pl.BlockSpec(bs, lambda i: (i, 0))
    row3 = pl.BlockSpec((2, B, 32), lambda i: (0, i, 0))
    full = lambda bs: pl.BlockSpec(bs, lambda i: (0, 0))
    in_specs = [row((B, 64))] + [full((64, 64)), full((64, 1))] * (ns + nd)
    out_specs = ([row3, row((B, 1))] * ns + [row((B, 1))] * nd)
    out_shape = ([jax.ShapeDtypeStruct((2, n, 32), F32),
                  jax.ShapeDtypeStruct((n, 1), F32)] * ns
                 + [jax.ShapeDtypeStruct((n, 1), F32)] * nd)
    args = [x]
    for p in src_params:
        args += [p['W_src'], p['att_src'].reshape(64, 1)]
    for p in dst_params:
        args += [p['W_dst'], p['att_dst'].reshape(64, 1)]
    return pl.pallas_call(
        body, grid=(n // B,), in_specs=in_specs, out_specs=out_specs,
        out_shape=out_shape)(*args)


def _combine(n, contribs, biases, relu=False, score_w=None, score_b=None):
    B = 1000
    k = len(contribs)

    def body(*refs):
        ins = refs[:-1]
        o_ref = refs[-1]
        o_lo = jnp.zeros((B, 32), F32)
        o_hi = jnp.zeros((B, 32), F32)
        for i in range(k):
            nlo, nhi, dlo, dhi = ins[4 * i:4 * i + 4]
            o_lo = o_lo + nlo[...] / (dlo[...] + EPS)
            o_hi = o_hi + nhi[...] / (dhi[...] + EPS)
        bias = ins[4 * k][...]
        for i in range(1, k):
            bias = bias + ins[4 * k + i][...]
        o_lo = o_lo + bias[:, :32]
        o_hi = o_hi + bias[:, 32:]
        if relu:
            o_ref[...] = jnp.concatenate(
                [jnp.maximum(o_lo, 0.0), jnp.maximum(o_hi, 0.0)], axis=1)
        else:
            w = ins[4 * k + k][...]
            b = ins[4 * k + k + 1][...]
            o_ref[...] = (jnp.dot(o_lo, w[:32], preferred_element_type=F32)
                          + jnp.dot(o_hi, w[32:], preferred_element_type=F32)
                          + b)

    row = lambda bs: pl.BlockSpec(bs, lambda i: (i, 0))
    full = lambda bs: pl.BlockSpec(bs, lambda i: (0, 0))
    in_specs = []
    args = []
    for (nlo, nhi, dlo, dhi) in contribs:
        in_specs += [row((B, 32)), row((B, 32)), row((B, 1)), row((B, 1))]
        args += [nlo, nhi, dlo.reshape(n, 1), dhi.reshape(n, 1)]
    for b in biases:
        in_specs.append(full((1, 64)))
        args.append(b.reshape(1, 64))
    if score_w is None:
        out_shape = jax.ShapeDtypeStruct((n, 64), F32)
        out_spec = row((B, 64))
    else:
        in_specs += [full((64, 1)), full((1, 1))]
        args += [score_w.reshape(64, 1), score_b.reshape(1, 1)]
        out_shape = jax.ShapeDtypeStruct((n, 1), F32)
        out_spec = row((B, 1))
    return pl.pallas_call(
        body, grid=(n // B,), in_specs=in_specs, out_specs=out_spec,
        out_shape=out_shape)(*args)


@functools.lru_cache(maxsize=None)
def _edge_pass(n_src, n_dst):
    ept = E // NS
    shared_words = 34 * n_dst + n_src
    BGB = 2000 if shared_words + 16 * 37 * 2000 < 2_000_000 else 400
    nblk = ept // BGB
    nch = n_dst // BGB
    mesh = plsc.VectorSubcoreMesh(core_axis_name="c", subcore_axis_name="s")

    @functools.partial(
        pl.kernel,
        out_type=[jax.ShapeDtypeStruct((n_dst, 32), F32),
                  jax.ShapeDtypeStruct((n_dst, 32), F32),
                  jax.ShapeDtypeStruct((n_dst,), F32),
                  jax.ShapeDtypeStruct((n_dst,), F32)],
        mesh=mesh,
        scratch_types=[
            pltpu.VMEM((BGB,), jnp.int32),
            pltpu.VMEM((BGB,), jnp.int32),
            pltpu.VMEM((BGB, 32), F32),
            pltpu.VMEM((BGB,), F32),
            pltpu.VMEM((BGB,), F32),
            pltpu.VMEM((BGB,), F32),
            pltpu.VMEM_SHARED((n_src,), F32),
            pltpu.VMEM_SHARED((n_dst,), F32),
            pltpu.VMEM_SHARED((n_dst, 32), F32),
            pltpu.VMEM_SHARED((n_dst,), F32),
            pltpu.SemaphoreType.DMA,
            pltpu.SemaphoreType.DMA,
        ],
        compiler_params=pltpu.CompilerParams(needs_layout_passes=False,
                                             use_tc_tiling_on_sc=False),
    )
    def k(src2_hbm, dst2_hbm, asrc_hbm, adst_hbm, hlo_hbm, hhi_hbm,
          numlo_out, numhi_out, denlo_out, denhi_out,
          sidx, didx, hrows, exb, asb, adb,
          asrc_sh, adst_sh, num_sh, den_sh, sem, sem2):
        cid = lax.axis_index("c")
        sid = lax.axis_index("s")

        @pl.when(sid == 0)
        def _():
            pltpu.sync_copy(asrc_hbm, asrc_sh)
            pltpu.sync_copy(adst_hbm, adst_sh)

        def zrow(i, c):
            hrows[i, pl.ds(0, 16)] = jnp.zeros((16,), F32)
            hrows[i, pl.ds(16, 16)] = jnp.zeros((16,), F32)
            return c

        lax.fori_loop(0, BGB, zrow, 0)

        def zex(i, c):
            exb[pl.ds(i * 16, 16)] = jnp.zeros((16,), F32)
            return c

        lax.fori_loop(0, BGB // 16, zex, 0)

        def zchunk(j, c):
            ch = j * NS + sid

            @pl.when(ch < nch)
            def _():
                sl = pl.ds(ch * BGB, BGB)
                pltpu.sync_copy(hrows, num_sh.at[sl])
                pltpu.sync_copy(exb, den_sh.at[sl])
            return c

        lax.fori_loop(0, _cdiv(nch, NS), zchunk, 0)
        plsc.subcore_barrier()

        HB = BGB // 2

        def eblk(b, c):
            off = sid * ept + b * BGB
            pltpu.sync_copy(src2_hbm.at[pl.ds(off, BGB)], sidx)
            pltpu.sync_copy(dst2_hbm.at[pl.ds(off, BGB)], didx)

            dc0 = [None, None]
            dc1 = [None, None]

            @pl.when(cid == 0)
            def _():
                dc0[0] = pltpu.async_copy(hlo_hbm.at[sidx.at[pl.ds(0, HB)]],
                                          hrows.at[pl.ds(0, HB)], sem)
                dc0[1] = pltpu.async_copy(hlo_hbm.at[sidx.at[pl.ds(HB, HB)]],
                                          hrows.at[pl.ds(HB, HB)], sem)

            @pl.when(cid == 1)
            def _():
                dc1[0] = pltpu.async_copy(hhi_hbm.at[sidx.at[pl.ds(0, HB)]],
                                          hrows.at[pl.ds(0, HB)], sem)
                dc1[1] = pltpu.async_copy(hhi_hbm.at[sidx.at[pl.ds(HB, HB)]],
                                          hrows.at[pl.ds(HB, HB)], sem)

            pltpu.async_copy(asrc_sh.at[sidx], asb, sem2).wait()
            pltpu.async_copy(adst_sh.at[didx], adb, sem2).wait()

            def cex(g, c2):
                av = asb[pl.ds(g * 16, 16)] + adb[pl.ds(g * 16, 16)]
                av = jnp.where(av >= 0.0, av, av * 0.2)
                exb[pl.ds(g * 16, 16)] = jnp.exp(av)
                return c2

            lax.fori_loop(0, BGB // 16, cex, 0)

            def srow(r0, c2):
                for u in range(4):
                    r = r0 * 4 + u
                    w = plsc.load_gather(exb, [jnp.broadcast_to(r, (16,))])
                    hrows[r, pl.ds(0, 16)] = hrows[r, pl.ds(0, 16)] * w
                    hrows[r, pl.ds(16, 16)] = hrows[r, pl.ds(16, 16)] * w
                return c2

            @pl.when(cid == 0)
            def _():
                dc0[0].wait()

            @pl.when(cid == 1)
            def _():
                dc1[0].wait()

            lax.fori_loop(0, HB // 4, srow, 0)

            @pl.when(cid == 0)
            def _():
                dc0[1].wait()

            @pl.when(cid == 1)
            def _():
                dc1[1].wait()

            lax.fori_loop(HB // 4, BGB // 4, srow, 0)
            pltpu.sync_copy(hrows, num_sh.at[didx], add=True)
            pltpu.sync_copy(exb, den_sh.at[didx], add=True)
            return c

        lax.fori_loop(0, nblk, eblk, 0)
        plsc.subcore_barrier()

        def rchunk(j, c):
            ch = j * NS + sid

            @pl.when(ch < nch)
            def _():
                sl = pl.ds(ch * BGB, BGB)

                @pl.when(cid == 0)
                def _():
                    pltpu.sync_copy(num_sh.at[sl], numlo_out.at[sl])
                    pltpu.sync_copy(den_sh.at[sl], denlo_out.at[sl])

                @pl.when(cid == 1)
                def _():
                    pltpu.sync_copy(num_sh.at[sl], numhi_out.at[sl])
                    pltpu.sync_copy(den_sh.at[sl], denhi_out.at[sl])
            return c

        lax.fori_loop(0, _cdiv(nch, NS), rchunk, 0)

    return k


@functools.lru_cache(maxsize=None)
def _score_kernel():
    nch = L_LBL // GB
    nw = NC * NS
    mesh = plsc.VectorSubcoreMesh(core_axis_name="c", subcore_axis_name="s")

    @functools.partial(
        pl.kernel,
        out_type=jax.ShapeDtypeStruct((L_LBL,), F32),
        mesh=mesh,
        scratch_types=[
            pltpu.VMEM((N_P,), F32),
            pltpu.VMEM((N_M,), F32),
            pltpu.VMEM((GB,), jnp.int32),
            pltpu.VMEM((GB,), jnp.int32),
            pltpu.VMEM((GB,), F32),
        ],
        compiler_params=pltpu.CompilerParams(needs_layout_passes=False),
    )
    def k(row_hbm, col_hbm, sp_hbm, sm_hbm, out_hbm, sp_t, sm_t, ridx, cidx, ob):
        wid = lax.axis_index("s") * NC + lax.axis_index("c")
        pltpu.sync_copy(sp_hbm, sp_t)
        pltpu.sync_copy(sm_hbm, sm_t)

        def chunk(j, c):
            ch = j * nw + wid

            @pl.when(ch < nch)
            def _():
                sl = pl.ds(ch * GB, GB)
                pltpu.sync_copy(row_hbm.at[sl], ridx)
                pltpu.sync_copy(col_hbm.at[sl], cidx)
                for g in range(GB // 16):
                    r16 = ridx[pl.ds(g * 16, 16)]
                    c16 = cidx[pl.ds(g * 16, 16)]
                    ob[pl.ds(g * 16, 16)] = (plsc.load_gather(sp_t, [r16])
                                             + plsc.load_gather(sm_t, [c16]))
                pltpu.sync_copy(ob, out_hbm.at[sl])
            return c

        lax.fori_loop(0, _cdiv(nch, nw), chunk, 0)

    return k


def _gat_layer(x, edges, pc, layer2):
    p_src_roles = [pc['pd'], pc['pm']] if not layer2 else [pc['pm']]
    p_dst_roles = [pc['dp'], pc['mp']]
    d_src_roles = [pc['dp']]
    d_dst_roles = [pc['pd']] if not layer2 else []
    m_src_roles = [pc['mp']]
    m_dst_roles = [pc['pm']]

    outp = _node_dense(x['patient'], p_src_roles, p_dst_roles)
    outd = _node_dense(x['disease'], d_src_roles, d_dst_roles)
    outm = _node_dense(x['medicine'], m_src_roles, m_dst_roles)

    if not layer2:
        h_pd = (outp[0], outp[1])
        h_pm = (outp[2], outp[3])
        a_dp_dst, a_mp_dst = outp[4], outp[5]
        h_dp = (outd[0], outd[1])
        a_pd_dst = outd[2]
        h_mp = (outm[0], outm[1])
        a_pm_dst = outm[2]
    else:
        h_pm = (outp[0], outp[1])
        a_dp_dst, a_mp_dst = outp[2], outp[3]
        h_dp = (outd[0], outd[1])
        h_mp = (outm[0], outm[1])
        a_pm_dst = outm[2]

    def run(et, h, a_dst, n_src, n_dst):
        return _edge_pass(n_src, n_dst)(
            edges[et][0], edges[et][1], h[1].reshape(n_src),
            a_dst.reshape(n_dst), h[0][0], h[0][1])

    r_dp = run('dp', h_dp, a_dp_dst, N_D, N_P)
    r_mp = run('mp', h_mp, a_mp_dst, N_M, N_P)
    r_pm = run('pm', h_pm, a_pm_dst, N_P, N_M)
    r_pd = None if layer2 else run('pd', h_pd, a_pd_dst, N_P, N_D)
    return r_dp, r_mp, r_pm, r_pd


def kernel(edge_index_pd, edge_index_dp, edge_index_pm, edge_index_mp,
           edge_label_index, params):
    edges = {'pd': edge_index_pd, 'dp': edge_index_dp,
             'pm': edge_index_pm, 'mp': edge_index_mp}
    pc1, pc2 = params['conv1'], params['conv2']

    x0 = params['emb']
    r_dp, r_mp, r_pm, r_pd = _gat_layer(x0, edges, pc1, layer2=False)
    x1 = {
        'patient': _combine(N_P, [r_dp, r_mp],
                            [pc1['dp']['bias'], pc1['mp']['bias']], relu=True),
        'disease': _combine(N_D, [r_pd], [pc1['pd']['bias']], relu=True),
        'medicine': _combine(N_M, [r_pm], [pc1['pm']['bias']], relu=True),
    }

    r_dp2, r_mp2, r_pm2, _ = _gat_layer(x1, edges, pc2, layer2=True)
    w = params['lin_W']
    s_p = _combine(N_P, [r_dp2, r_mp2],
                   [pc2['dp']['bias'], pc2['mp']['bias']],
                   score_w=w[:64, 0], score_b=params['lin_b'])
    s_m = _combine(N_M, [r_pm2], [pc2['pm']['bias']],
                   score_w=w[64:, 0], score_b=jnp.zeros((1,), F32))

    return _score_kernel()(edge_label_index[0], edge_label_index[1],
                           s_p.reshape(N_P), s_m.reshape(N_M))
